# Initial kernel scaffold; baseline (speedup 1.0000x reference)
import jax
import jax.numpy as jnp
from jax import lax
from jax.experimental import pallas as pl
from jax.experimental.pallas import tpu as pltpu
from jax.experimental.pallas import tpu_sc as plsc
from jax._src.pallas import mpmd as _mpmd

NUM_ASSETS = 10000
NUM_PORTFOLIOS = 4096
N_PAIRS = 204800

_NC = 2
_NS = 16
_NW = _NC * _NS
_PPW = N_PAIRS // _NW
_DMA_W = 128
_NDMA = _PPW // _DMA_W
_LANES = 16
_ROWS_BLK = 128


def _bcast_body(prior_ref, out_ref):
    out_ref[...] = jnp.broadcast_to(prior_ref[...], out_ref.shape)


def _scatter_body(dense_hbm, ai_hbm, pi_hbm, prior_hbm, out_hbm,
                  ai_v, pi_v, prior_v, idx_v, val_v, sem):
    del dense_hbm
    wid = lax.axis_index("s") * _NC + lax.axis_index("c")
    base = wid * _PPW
    pltpu.sync_copy(ai_hbm.at[pl.ds(base, _PPW)], ai_v)
    pltpu.sync_copy(pi_hbm.at[pl.ds(base, _PPW)], pi_v)
    pltpu.sync_copy(prior_hbm, prior_v)

    def compute_row(j, _):
        for c in range(_DMA_W // _LANES):
            off = j * _DMA_W + c * _LANES
            a = ai_v[pl.ds(off, _LANES)]
            p = pi_v[pl.ds(off, _LANES)]
            pv = plsc.load_gather(prior_v, [a])
            val_v[j, pl.ds(c * _LANES, _LANES)] = pv + 1.0
            idx_v[j, pl.ds(c * _LANES, _LANES)] = p * NUM_ASSETS + a
        return 0

    lax.fori_loop(0, _NDMA, compute_row, 0)

    def fire(j, _):
        pltpu.make_async_copy(val_v.at[j], out_hbm.at[idx_v.at[j]], sem).start()
        return 0

    def drain(j, _):
        pltpu.make_async_copy(val_v.at[j], out_hbm.at[idx_v.at[j]], sem).wait()
        return 0

    lax.fori_loop(0, _NDMA, fire, 0)
    lax.fori_loop(0, _NDMA, drain, 0)


def kernel(asset_indices, portfolio_indices, asset_prior_evidence):
    dense = pl.pallas_call(
        _bcast_body,
        grid=(NUM_PORTFOLIOS // _ROWS_BLK,),
        in_specs=[pl.BlockSpec((1, NUM_ASSETS), lambda i: (0, 0))],
        out_specs=pl.BlockSpec((_ROWS_BLK, NUM_ASSETS), lambda i: (i, 0)),
        out_shape=jax.ShapeDtypeStruct(
            (NUM_PORTFOLIOS, NUM_ASSETS), jnp.float32),
    )(asset_prior_evidence.reshape(1, NUM_ASSETS))

    mesh = plsc.VectorSubcoreMesh(core_axis_name="c", subcore_axis_name="s")
    scatter = _mpmd._mpmd_map(
        [(mesh, _scatter_body)],
        out_types=jax.ShapeDtypeStruct(
            (NUM_PORTFOLIOS * NUM_ASSETS,), jnp.float32),
        input_output_aliases={0: 0},
        scratch_types=[
            pltpu.VMEM((_PPW,), jnp.int32),
            pltpu.VMEM((_PPW,), jnp.int32),
            pltpu.VMEM((NUM_ASSETS,), jnp.float32),
            pltpu.VMEM((_NDMA, _DMA_W), jnp.int32),
            pltpu.VMEM((_NDMA, _DMA_W), jnp.float32),
            pltpu.SemaphoreType.DMA,
        ],
    )
    out_flat = scatter(
        dense.reshape(-1), asset_indices, portfolio_indices,
        asset_prior_evidence)
    return out_flat.reshape(NUM_PORTFOLIOS, NUM_ASSETS)


# speedup vs baseline: 1.8340x; 1.8340x over previous
import jax
import jax.numpy as jnp
from jax import lax
from jax.experimental import pallas as pl
from jax.experimental.pallas import tpu as pltpu
from jax.experimental.pallas import tpu_sc as plsc
from jax._src.pallas import mpmd as _mpmd

NUM_ASSETS = 10000
NUM_PORTFOLIOS = 4096
N_PAIRS = 204800

_NC = 2
_NS = 16
_NW = _NC * _NS
_PPW = N_PAIRS // _NW
_DMA_W = 128
_NDMA = _PPW // _DMA_W
_LANES = 16
_ROWS_BLK = 128


def _bcast_body(prior_ref, out_ref):
    out_ref[...] = jnp.broadcast_to(prior_ref[...], out_ref.shape)


def _scatter_body(dense_hbm, ai_hbm, pi_hbm, prior_hbm, out_hbm,
                  ai_v, pi_v, prior_v, idx_v, val_v, sem):
    del dense_hbm
    wid = lax.axis_index("s") * _NC + lax.axis_index("c")
    base = wid * _PPW
    pltpu.sync_copy(ai_hbm.at[pl.ds(base, _PPW)], ai_v)
    pltpu.sync_copy(pi_hbm.at[pl.ds(base, _PPW)], pi_v)
    pltpu.sync_copy(prior_hbm, prior_v)

    def compute_row(j, _):
        for c in range(_DMA_W // _LANES):
            off = j * _DMA_W + c * _LANES
            a = ai_v[pl.ds(off, _LANES)]
            p = pi_v[pl.ds(off, _LANES)]
            pv = plsc.load_gather(prior_v, [a])
            val_v[j, pl.ds(c * _LANES, _LANES)] = pv + 1.0
            idx_v[j, pl.ds(c * _LANES, _LANES)] = p * NUM_ASSETS + a
        return 0

    lax.fori_loop(0, _NDMA, compute_row, 0)

    def fire(j, _):
        pltpu.make_async_copy(val_v.at[j], out_hbm.at[idx_v.at[j]], sem).start()
        return 0

    def drain(j, _):
        pltpu.make_async_copy(val_v.at[j], out_hbm.at[idx_v.at[j]], sem).wait()
        return 0

    lax.fori_loop(0, _NDMA, fire, 0)
    lax.fori_loop(0, _NDMA, drain, 0)


def kernel(asset_indices, portfolio_indices, asset_prior_evidence):
    dense = pl.pallas_call(
        _bcast_body,
        grid=(NUM_PORTFOLIOS // _ROWS_BLK,),
        in_specs=[pl.BlockSpec((1, NUM_ASSETS), lambda i: (0, 0))],
        out_specs=pl.BlockSpec((_ROWS_BLK, NUM_ASSETS), lambda i: (i, 0)),
        out_shape=jax.ShapeDtypeStruct(
            (NUM_PORTFOLIOS, NUM_ASSETS), jnp.float32),
    )(asset_prior_evidence.reshape(1, NUM_ASSETS))

    mesh = plsc.VectorSubcoreMesh(core_axis_name="c", subcore_axis_name="s")
    scatter = _mpmd._mpmd_map(
        [(mesh, _scatter_body)],
        out_types=jax.ShapeDtypeStruct(
            (NUM_PORTFOLIOS * NUM_ASSETS,), jnp.float32),
        input_output_aliases={0: 0},
        compiler_params=pltpu.CompilerParams(needs_layout_passes=False),
        scratch_types=[
            pltpu.VMEM((_PPW,), jnp.int32),
            pltpu.VMEM((_PPW,), jnp.int32),
            pltpu.VMEM((NUM_ASSETS,), jnp.float32),
            pltpu.VMEM((_NDMA, _DMA_W), jnp.int32),
            pltpu.VMEM((_NDMA, _DMA_W), jnp.float32),
            pltpu.SemaphoreType.DMA,
        ],
    )
    out_flat = scatter(
        dense.reshape(-1), asset_indices, portfolio_indices,
        asset_prior_evidence)
    return out_flat.reshape(NUM_PORTFOLIOS, NUM_ASSETS)


# speedup vs baseline: 2.3163x; 1.2630x over previous
import jax
import jax.numpy as jnp
from jax import lax
from jax.experimental import pallas as pl
from jax.experimental.pallas import tpu as pltpu
from jax.experimental.pallas import tpu_sc as plsc
from jax._src.pallas import mpmd as _mpmd

NUM_ASSETS = 10000
NUM_PORTFOLIOS = 4096
N_PAIRS = 204800

_NC = 2
_NS = 16
_NW = _NC * _NS
_PPW = N_PAIRS // _NW
_DMA_W = 128
_NDMA = _PPW // _DMA_W
_LANES = 16
_ROWS_BLK = 128


_ROWS_PER_W = NUM_PORTFOLIOS // _NW


def _dense_body(prior_hbm, out_hbm, prior_v, sem):
    wid = lax.axis_index("s") * _NC + lax.axis_index("c")
    row0 = wid * _ROWS_PER_W
    pltpu.sync_copy(prior_hbm, prior_v)

    def fire(r, _):
        pltpu.make_async_copy(
            prior_v, out_hbm.at[pl.ds((row0 + r) * NUM_ASSETS, NUM_ASSETS)],
            sem).start()
        return 0

    def drain(r, _):
        pltpu.make_async_copy(
            prior_v, out_hbm.at[pl.ds((row0 + r) * NUM_ASSETS, NUM_ASSETS)],
            sem).wait()
        return 0

    lax.fori_loop(0, _ROWS_PER_W, fire, 0)
    lax.fori_loop(0, _ROWS_PER_W, drain, 0)


def _scatter_body(dense_hbm, ai_hbm, pi_hbm, prior_hbm, out_hbm,
                  ai_v, pi_v, prior_v, idx_v, val_v, sem):
    del dense_hbm
    wid = lax.axis_index("s") * _NC + lax.axis_index("c")
    base = wid * _PPW
    pltpu.sync_copy(ai_hbm.at[pl.ds(base, _PPW)], ai_v)
    pltpu.sync_copy(pi_hbm.at[pl.ds(base, _PPW)], pi_v)
    pltpu.sync_copy(prior_hbm, prior_v)

    def compute_row(j, _):
        for c in range(_DMA_W // _LANES):
            off = j * _DMA_W + c * _LANES
            a = ai_v[pl.ds(off, _LANES)]
            p = pi_v[pl.ds(off, _LANES)]
            pv = plsc.load_gather(prior_v, [a])
            val_v[j, pl.ds(c * _LANES, _LANES)] = pv + 1.0
            idx_v[j, pl.ds(c * _LANES, _LANES)] = p * NUM_ASSETS + a
        return 0

    lax.fori_loop(0, _NDMA, compute_row, 0)

    def fire(j, _):
        pltpu.make_async_copy(val_v.at[j], out_hbm.at[idx_v.at[j]], sem).start()
        return 0

    def drain(j, _):
        pltpu.make_async_copy(val_v.at[j], out_hbm.at[idx_v.at[j]], sem).wait()
        return 0

    lax.fori_loop(0, _NDMA, fire, 0)
    lax.fori_loop(0, _NDMA, drain, 0)


def kernel(asset_indices, portfolio_indices, asset_prior_evidence):
    mesh = plsc.VectorSubcoreMesh(core_axis_name="c", subcore_axis_name="s")
    dense_fn = _mpmd._mpmd_map(
        [(mesh, _dense_body)],
        out_types=jax.ShapeDtypeStruct(
            (NUM_PORTFOLIOS * NUM_ASSETS,), jnp.float32),
        compiler_params=pltpu.CompilerParams(needs_layout_passes=False),
        scratch_types=[
            pltpu.VMEM((NUM_ASSETS,), jnp.float32),
            pltpu.SemaphoreType.DMA,
        ],
    )
    dense = dense_fn(asset_prior_evidence)
    scatter = _mpmd._mpmd_map(
        [(mesh, _scatter_body)],
        out_types=jax.ShapeDtypeStruct(
            (NUM_PORTFOLIOS * NUM_ASSETS,), jnp.float32),
        input_output_aliases={0: 0},
        compiler_params=pltpu.CompilerParams(needs_layout_passes=False),
        scratch_types=[
            pltpu.VMEM((_PPW,), jnp.int32),
            pltpu.VMEM((_PPW,), jnp.int32),
            pltpu.VMEM((NUM_ASSETS,), jnp.float32),
            pltpu.VMEM((_NDMA, _DMA_W), jnp.int32),
            pltpu.VMEM((_NDMA, _DMA_W), jnp.float32),
            pltpu.SemaphoreType.DMA,
        ],
    )
    out_flat = scatter(
        dense, asset_indices, portfolio_indices, asset_prior_evidence)
    return out_flat.reshape(NUM_PORTFOLIOS, NUM_ASSETS)
